# Initial kernel scaffold; baseline (speedup 1.0000x reference)
import jax
import jax.numpy as jnp
from jax.experimental import pallas as pl


def kernel(specimen_ids, organism_ids, antibiotic_ids, interpretation_ids, specimen_table, organism_table, antibiotic_table, interpretation_table):
    raise NotImplementedError("write your pallas kernel here")


# speedup vs baseline: 6.1969x; 6.1969x over previous
import functools

import jax
import jax.numpy as jnp
from jax import lax
from jax.experimental import pallas as pl
from jax.experimental.pallas import tpu as pltpu
from jax.experimental.pallas import tpu_sc as plsc

HIDDEN = 128
_INFO = plsc.get_sparse_core_info()
NC = _INFO.num_cores
NS = _INFO.num_subcores
NW = NC * NS

C = 128
NCHUNK = 200
ISTAGE = 40
NSTAGE = NCHUNK // ISTAGE
N_TOTAL = NW * NCHUNK * C


def _sc_body(spec_idx, org_idx, abx_idx, intp_idx,
             spec_t, org_t, abx_t, intp_t,
             out_h,
             idx_s, idx_o, idx_a, idx_i,
             buf_s, buf_o, buf_a, buf_i,
             sem):
    wid = lax.axis_index("s") * NC + lax.axis_index("c")

    def stage_body(st, _):
        blk = pl.ds(st * ISTAGE, ISTAGE)
        pltpu.sync_copy(spec_idx.at[wid, blk], idx_s)
        pltpu.sync_copy(org_idx.at[wid, blk], idx_o)
        pltpu.sync_copy(abx_idx.at[wid, blk], idx_a)
        pltpu.sync_copy(intp_idx.at[wid, blk], idx_i)

        def chunk_body(j, _):
            h1 = pltpu.async_copy(spec_t.at[idx_s.at[j]], buf_s, sem)
            h2 = pltpu.async_copy(org_t.at[idx_o.at[j]], buf_o, sem)
            h3 = pltpu.async_copy(abx_t.at[idx_a.at[j]], buf_a, sem)
            h4 = pltpu.async_copy(intp_t.at[idx_i.at[j]], buf_i, sem)
            h1.wait()
            h2.wait()
            h3.wait()
            h4.wait()

            def row_body(r, _):
                for jj in range(HIDDEN // 16):
                    sl = pl.ds(jj * 16, 16)
                    buf_s[r, sl] = ((buf_s[r, sl] + buf_o[r, sl])
                                    + (buf_a[r, sl] + buf_i[r, sl]))
                return 0

            lax.fori_loop(0, C, row_body, 0)
            row0 = (wid * NCHUNK + st * ISTAGE + j) * C
            pltpu.sync_copy(buf_s, out_h.at[pl.ds(row0, C)])
            return 0

        lax.fori_loop(0, ISTAGE, chunk_body, 0)
        return 0

    lax.fori_loop(0, NSTAGE, stage_body, 0)


@functools.partial(
    pl.kernel,
    mesh=plsc.VectorSubcoreMesh(core_axis_name="c", subcore_axis_name="s"),
    out_type=jax.ShapeDtypeStruct((N_TOTAL, HIDDEN), jnp.float32),
    scratch_types=[
        pltpu.VMEM((ISTAGE, C), jnp.int32),
        pltpu.VMEM((ISTAGE, C), jnp.int32),
        pltpu.VMEM((ISTAGE, C), jnp.int32),
        pltpu.VMEM((ISTAGE, C), jnp.int32),
        pltpu.VMEM((C, HIDDEN), jnp.float32),
        pltpu.VMEM((C, HIDDEN), jnp.float32),
        pltpu.VMEM((C, HIDDEN), jnp.float32),
        pltpu.VMEM((C, HIDDEN), jnp.float32),
        pltpu.SemaphoreType.DMA,
    ],
)
def _embed_sum(spec_idx, org_idx, abx_idx, intp_idx,
               spec_t, org_t, abx_t, intp_t,
               out_h, *scratch):
    _sc_body(spec_idx, org_idx, abx_idx, intp_idx,
             spec_t, org_t, abx_t, intp_t, out_h, *scratch)


def kernel(specimen_ids, organism_ids, antibiotic_ids, interpretation_ids,
           specimen_table, organism_table, antibiotic_table, interpretation_table):
    batch, hist = specimen_ids.shape
    shp = (NW, NCHUNK, C)
    out = _embed_sum(
        specimen_ids.reshape(shp), organism_ids.reshape(shp),
        antibiotic_ids.reshape(shp), interpretation_ids.reshape(shp),
        specimen_table, organism_table, antibiotic_table, interpretation_table,
    )
    return out.reshape(batch, hist, HIDDEN)
